# baseline (speedup 1.0000x reference)
import functools

import jax
import jax.numpy as jnp
from jax import lax
from jax.experimental import pallas as pl
from jax.experimental.pallas import tpu as pltpu
from jax.experimental.pallas import tpu_sc as plsc

N = 10000
D = 256
LANE = 128
CHUNK = 128
TILES = 16
NP = 10240
NPB = 10048
TRASH = NPB - N
NT = NP

JITTER = 1e-20


def _edge_layout(E):
    ep = ((E + 32 * CHUNK - 1) // (32 * CHUNK)) * (32 * CHUNK)
    return ep, ep // CHUNK


_QROWS = NP // LANE


def _scatter_body(nch_t, g4_hbm, srcs_hbm, dst_hbm, out_hbm,
                  src_v, dst_v, buf0, buf1, acc, sem0, sem1):
    cid = lax.axis_index("c")
    sid = lax.axis_index("s")
    rows_t = 624
    tail = N - TILES * rows_t

    hb = nch_t // 2

    for h in range(2):
        ph = h * 2 + cid
        pltpu.sync_copy(g4_hbm.at[pl.ds(ph * NT + sid * rows_t, rows_t)],
                        acc.at[pl.ds(sid * rows_t, rows_t)])

        @pl.when(sid == TILES - 1)
        def _():
            pltpu.sync_copy(
                g4_hbm.at[pl.ds(ph * NT + TILES * rows_t, tail)],
                acc.at[pl.ds(TILES * rows_t, tail)])

        for half in range(2):
            base = sid * nch_t + half * hb
            pltpu.sync_copy(srcs_hbm.at[pl.ds(ph * (TILES * nch_t) + base, hb)],
                            src_v)
            pltpu.sync_copy(dst_hbm.at[pl.ds(base, hb)], dst_v)
            if half == 0:
                plsc.subcore_barrier()

            def gather(j, buf, sem):
                pltpu.async_copy(g4_hbm.at[src_v.at[j]], buf, sem)

            def gwait(j, buf, sem):
                pltpu.make_async_copy(g4_hbm.at[src_v.at[j]], buf, sem).wait()

            gather(0, buf0, sem0)
            gather(1, buf1, sem1)

            @pl.loop(0, hb // 2)
            def _(t):
                j = 2 * t
                gwait(j, buf0, sem0)
                pltpu.sync_copy(buf0, acc.at[dst_v.at[j]], add=True)

                @pl.when(j + 2 < hb)
                def _():
                    gather(j + 2, buf0, sem0)

                gwait(j + 1, buf1, sem1)
                pltpu.sync_copy(buf1, acc.at[dst_v.at[j + 1]], add=True)

                @pl.when(j + 3 < hb)
                def _():
                    gather(j + 3, buf1, sem1)

        plsc.subcore_barrier()
        pltpu.sync_copy(acc.at[pl.ds(sid * rows_t, rows_t)],
                        out_hbm.at[pl.ds(ph * N + sid * rows_t, rows_t)])

        @pl.when(sid == TILES - 1)
        def _():
            pltpu.sync_copy(
                acc.at[pl.ds(TILES * rows_t, tail)],
                out_hbm.at[pl.ds(ph * N + TILES * rows_t, tail)])

        plsc.subcore_barrier()


def _scatter_call(g4, srcs, dst2d, nch):
    nch_t = nch // TILES
    mesh = plsc.VectorSubcoreMesh(core_axis_name="c", subcore_axis_name="s")
    f = functools.partial(
        pl.kernel,
        out_type=jax.ShapeDtypeStruct((4 * N, LANE), jnp.float32),
        mesh=mesh,
        scratch_types=[
            pltpu.VMEM((nch_t // 2, CHUNK), jnp.int32),
            pltpu.VMEM((nch_t // 2, CHUNK), jnp.int32),
            pltpu.VMEM((CHUNK, LANE), jnp.float32),
            pltpu.VMEM((CHUNK, LANE), jnp.float32),
            pltpu.VMEM_SHARED((NPB, LANE), jnp.float32),
            pltpu.SemaphoreType.DMA,
            pltpu.SemaphoreType.DMA,
        ],
    )(functools.partial(_scatter_body, nch_t))
    return f(g4, srcs, dst2d)


_R = 2048
_G = NT // _R
_ESTEPS = 16


def _k1_body(ew, dst_ref, x_ref, wa_ref, wc_ref, g_ref, dinv_ref, accq):
    i = pl.program_id(0)

    @pl.when(i < _ESTEPS)
    def _():
        d = dst_ref[...][0]
        qv = d // LANE
        rv = d % LANE
        qoh = (jnp.broadcast_to(qv, (_QROWS, ew))
               == lax.broadcasted_iota(jnp.int32, (_QROWS, ew), 0)
               ).astype(jnp.bfloat16)
        roh = (jnp.broadcast_to(rv, (LANE, ew))
               == lax.broadcasted_iota(jnp.int32, (LANE, ew), 0)
               ).astype(jnp.bfloat16)
        p = lax.dot_general(qoh, roh, (((1,), (1,)), ((), ())),
                            preferred_element_type=jnp.float32)
        accq[...] = jnp.where(i == 0, p, accq[...] + p)

    @pl.when(i >= _ESTEPS)
    def _():
        j = i - _ESTEPS
        degblk = accq[pl.ds(j * (_R // LANE), _R // LANE), :]
        rep = (lax.broadcasted_iota(jnp.int32, (_R, _R // LANE), 0) // LANE
               == lax.broadcasted_iota(jnp.int32, (_R, _R // LANE), 1)
               ).astype(jnp.float32)
        rows = lax.dot_general(rep, degblk, (((1,), (0,)), ((), ())),
                               preferred_element_type=jnp.float32)
        lanemask = (lax.broadcasted_iota(jnp.int32, (_R, LANE), 0) % LANE
                    == lax.broadcasted_iota(jnp.int32, (_R, LANE), 1))
        deg = jnp.sum(jnp.where(lanemask, rows, 0.0), axis=1) + 1.0
        dv = lax.rsqrt(jnp.maximum(deg, 1e-12))
        dinv_ref[0, 0, :] = dv
        dinv = dv[:, None]
        xb = x_ref[...].astype(jnp.bfloat16)
        ha = jnp.dot(xb, wa_ref[...].astype(jnp.bfloat16),
                     preferred_element_type=jnp.float32)
        hc = jnp.dot(xb, wc_ref[...].astype(jnp.bfloat16),
                     preferred_element_type=jnp.float32)
        g_ref[0] = ha[:, :LANE] * dinv
        g_ref[1] = ha[:, LANE:] * dinv
        g_ref[2] = hc[:, :LANE] * dinv
        g_ref[3] = hc[:, LANE:] * dinv


def _k1_call(x, wa, wc, dst_rows):
    nsteps, _, ew = dst_rows.shape
    assert nsteps == _ESTEPS
    nj = lambda i: jnp.maximum(i - _ESTEPS, 0)
    return pl.pallas_call(
        functools.partial(_k1_body, ew),
        grid=(_ESTEPS + _G,),
        in_specs=[
            pl.BlockSpec((1, 1, ew), lambda i: (jnp.minimum(i, _ESTEPS - 1), 0, 0)),
            pl.BlockSpec((_R, D), lambda i: (nj(i), 0)),
            pl.BlockSpec((D, D), lambda i: (0, 0)),
            pl.BlockSpec((D, D), lambda i: (0, 0)),
        ],
        out_specs=[
            pl.BlockSpec((4, _R, LANE), lambda i: (0, nj(i), 0)),
            pl.BlockSpec((1, 1, _R), lambda i: (nj(i), 0, 0)),
        ],
        out_shape=[
            jax.ShapeDtypeStruct((4, NT, LANE), jnp.float32),
            jax.ShapeDtypeStruct((_G, 1, _R), jnp.float32),
        ],
        scratch_shapes=[pltpu.VMEM((_QROWS, LANE), jnp.float32)],
    )(dst_rows, x, wa, wc)


def _sigmoid(x):
    return 1.0 / (1.0 + jnp.exp(-x))


def _softplus(x):
    return jnp.maximum(x, 0.0) + jnp.log1p(jnp.exp(-jnp.abs(x)))


def _k2_body(s_ref, dinv_ref, x_ref, ba_ref, bc_ref,
             w1a_ref, b1a_ref, w2a_ref, b2a_ref, w3a_ref, b3a_ref,
             w1c_ref, b1c_ref, w2c_ref, b2c_ref, w3c_ref, b3c_ref,
             conc_ref, nz_ref, val_ref, pooled):
    i = pl.program_id(0)
    dinv = dinv_ref[0, 0, :][:, None]
    xb = x_ref[...]
    valid2 = (i * _R + lax.broadcasted_iota(jnp.int32, (_R, 1), 0)) < N

    sa = jnp.concatenate([s_ref[0], s_ref[1]], axis=1)
    ha = jnp.maximum(sa * dinv + ba_ref[...], 0.0) + xb
    t = jnp.maximum(jnp.dot(ha.astype(jnp.bfloat16),
                            w1a_ref[...].astype(jnp.bfloat16),
                            preferred_element_type=jnp.float32)
                    + b1a_ref[...], 0.0)
    t = jnp.maximum(jnp.dot(t.astype(jnp.bfloat16),
                            w2a_ref[...].astype(jnp.bfloat16),
                            preferred_element_type=jnp.float32)
                    + b2a_ref[...], 0.0)
    lg = jnp.dot(t.astype(jnp.bfloat16), w3a_ref[...].astype(jnp.bfloat16),
                 preferred_element_type=jnp.float32) + b3a_ref[...]
    conc_ref[0, 0, :] = _softplus(lg[:, 0]) + JITTER
    nz_ref[0, 0, :] = _sigmoid(lg[:, 1])

    sc = jnp.concatenate([s_ref[2], s_ref[3]], axis=1)
    hc = jnp.maximum(sc * dinv + bc_ref[...], 0.0) + xb
    ps = jnp.sum(jnp.where(valid2, hc, 0.0), axis=0)
    pooled[0, :] = jnp.where(i == 0, ps, pooled[0, :] + ps)

    @pl.when(i == _G - 1)
    def _():
        p = pooled[0, :][None, :]
        u = jnp.maximum(jnp.dot(p, w1c_ref[...], preferred_element_type=jnp.float32)
                        + b1c_ref[...], 0.0)
        u = jnp.maximum(jnp.dot(u, w2c_ref[...], preferred_element_type=jnp.float32)
                        + b2c_ref[...], 0.0)
        v = jnp.dot(u, w3c_ref[...], preferred_element_type=jnp.float32)
        val_ref[...] = v[:, :1] + b3c_ref[...]


def _k2_call(s4, dinv5, x, ba, bc, w1a, b1a, w2a, b2a, w3ap, b3ap,
             w1c, b1c, w2c, b2c, w3cp, b3c2):
    full = lambda shape: pl.BlockSpec(shape, lambda i: tuple(0 for _ in shape))
    return pl.pallas_call(
        _k2_body,
        grid=(_G,),
        in_specs=[
            pl.BlockSpec((4, _R, LANE), lambda i: (0, i, 0)),
            pl.BlockSpec((1, 1, _R), lambda i: (i, 0, 0)),
            pl.BlockSpec((_R, D), lambda i: (i, 0)),
            full((D,)), full((D,)),
            full((D, 128)), full((128,)), full((128, 64)), full((64,)),
            full((64, 128)), full((128,)),
            full((D, 128)), full((128,)), full((128, 64)), full((64,)),
            full((64, 128)), full((1, 1)),
        ],
        out_specs=[
            pl.BlockSpec((1, 1, _R), lambda i: (i, 0, 0)),
            pl.BlockSpec((1, 1, _R), lambda i: (i, 0, 0)),
            pl.BlockSpec((1, 1), lambda i: (0, 0)),
        ],
        out_shape=[
            jax.ShapeDtypeStruct((_G, 1, _R), jnp.float32),
            jax.ShapeDtypeStruct((_G, 1, _R), jnp.float32),
            jax.ShapeDtypeStruct((1, 1), jnp.float32),
        ],
        scratch_shapes=[pltpu.VMEM((1, D), jnp.float32)],
    )(s4, dinv5, x, ba, bc, w1a, b1a, w2a, b2a, w3ap, b3ap,
      w1c, b1c, w2c, b2c, w3cp, b3c2)


def kernel(x, edge_index, W_gcn_a, b_gcn_a, W1a, b1a, W2a, b2a, W3a, b3a,
           W_gcn_c, b_gcn_c, W1c, b1c, W2c, b2c, W3c, b3c):
    E = edge_index.shape[1]
    ep, nch = _edge_layout(E)
    pad = ep - E

    ei = edge_index.astype(jnp.int32)
    apad = jnp.arange(pad, dtype=jnp.int32)
    src_p = jnp.concatenate([ei[0], apad % N])
    dst_p = jnp.concatenate([ei[1], N + apad % TRASH])
    srcs = (src_p[None, :]
            + (jnp.arange(4, dtype=jnp.int32) * NT)[:, None]).reshape(-1, CHUNK)
    dst2d = dst_p.reshape(nch, CHUNK)

    g4, dinv5 = _k1_call(x, W_gcn_a, W_gcn_c, dst_p.reshape(_ESTEPS, 1, -1))
    g4 = g4.reshape(4 * NT, LANE)
    s4 = _scatter_call(g4, srcs, dst2d, nch).reshape(4, N, LANE)

    w3ap = jnp.pad(W3a, ((0, 0), (0, 126)))
    b3ap = jnp.pad(b3a, (0, 126))
    w3cp = jnp.pad(W3c, ((0, 0), (0, 127)))
    b3c2 = b3c.reshape(1, 1)
    conc2, nz2, val = _k2_call(
        s4, dinv5, x, b_gcn_a, b_gcn_c, W1a, b1a, W2a, b2a, w3ap, b3ap,
        W1c, b1c, W2c, b2c, w3cp, b3c2)
    return conc2.reshape(-1)[:N], nz2.reshape(-1)[:N], val.reshape(-1)


# speedup vs baseline: 1.0040x; 1.0040x over previous
import functools

import jax
import jax.numpy as jnp
from jax import lax
from jax.experimental import pallas as pl
from jax.experimental.pallas import tpu as pltpu
from jax.experimental.pallas import tpu_sc as plsc

N = 10000
D = 256
LANE = 128
CHUNK = 128
TILES = 16
NP = 10240
NPB = 10048
TRASH = NPB - N
NT = NP

JITTER = 1e-20


def _edge_layout(E):
    ep = ((E + 32 * CHUNK - 1) // (32 * CHUNK)) * (32 * CHUNK)
    return ep, ep // CHUNK


_QROWS = NP // LANE


def _scatter_body(nch_t, g4_hbm, srcs_hbm, dst_hbm, out_hbm,
                  src_v, dst_v, buf0, buf1, acc, sem0, sem1):
    cid = lax.axis_index("c")
    sid = lax.axis_index("s")
    rows_t = 624
    tail = N - TILES * rows_t

    hb = nch_t // 2

    for h in range(2):
        ph = h * 2 + cid
        pltpu.sync_copy(g4_hbm.at[pl.ds(ph * NT + sid * rows_t, rows_t)],
                        acc.at[pl.ds(sid * rows_t, rows_t)])

        @pl.when(sid == TILES - 1)
        def _():
            pltpu.sync_copy(
                g4_hbm.at[pl.ds(ph * NT + TILES * rows_t, tail)],
                acc.at[pl.ds(TILES * rows_t, tail)])

        for half in range(2):
            base = sid * nch_t + half * hb
            pltpu.sync_copy(srcs_hbm.at[pl.ds(ph * (TILES * nch_t) + base, hb)],
                            src_v)
            pltpu.sync_copy(dst_hbm.at[pl.ds(base, hb)], dst_v)
            if half == 0:
                plsc.subcore_barrier()

            def gather(j, buf, sem):
                pltpu.async_copy(g4_hbm.at[src_v.at[j]], buf, sem)

            def gwait(j, buf, sem):
                pltpu.make_async_copy(g4_hbm.at[src_v.at[j]], buf, sem).wait()

            gather(0, buf0, sem0)
            gather(1, buf1, sem1)

            @pl.loop(0, hb // 2)
            def _(t):
                j = 2 * t
                gwait(j, buf0, sem0)
                pltpu.sync_copy(buf0, acc.at[dst_v.at[j]], add=True)

                @pl.when(j + 2 < hb)
                def _():
                    gather(j + 2, buf0, sem0)

                gwait(j + 1, buf1, sem1)
                pltpu.sync_copy(buf1, acc.at[dst_v.at[j + 1]], add=True)

                @pl.when(j + 3 < hb)
                def _():
                    gather(j + 3, buf1, sem1)

        plsc.subcore_barrier()
        pltpu.sync_copy(acc.at[pl.ds(sid * rows_t, rows_t)],
                        out_hbm.at[pl.ds(ph * N + sid * rows_t, rows_t)])

        @pl.when(sid == TILES - 1)
        def _():
            pltpu.sync_copy(
                acc.at[pl.ds(TILES * rows_t, tail)],
                out_hbm.at[pl.ds(ph * N + TILES * rows_t, tail)])

        plsc.subcore_barrier()


def _scatter_call(g4, srcs, dst2d, nch):
    nch_t = nch // TILES
    mesh = plsc.VectorSubcoreMesh(core_axis_name="c", subcore_axis_name="s")
    f = functools.partial(
        pl.kernel,
        out_type=jax.ShapeDtypeStruct((4 * N, LANE), jnp.float32),
        mesh=mesh,
        scratch_types=[
            pltpu.VMEM((nch_t // 2, CHUNK), jnp.int32),
            pltpu.VMEM((nch_t // 2, CHUNK), jnp.int32),
            pltpu.VMEM((CHUNK, LANE), jnp.float32),
            pltpu.VMEM((CHUNK, LANE), jnp.float32),
            pltpu.VMEM_SHARED((NPB, LANE), jnp.float32),
            pltpu.SemaphoreType.DMA,
            pltpu.SemaphoreType.DMA,
        ],
    )(functools.partial(_scatter_body, nch_t))
    return f(g4, srcs, dst2d)


_R = 2048
_G = NT // _R
_ESTEPS = 16


def _k1_body(ew, dst_ref, x_ref, wa_ref, wc_ref, g_ref, dinv_ref, accq):
    i = pl.program_id(0)

    @pl.when(i < _ESTEPS)
    def _():
        d = dst_ref[...][0]
        qv = d // LANE
        rv = d % LANE
        qoh = (jnp.broadcast_to(qv, (_QROWS, ew))
               == lax.broadcasted_iota(jnp.int32, (_QROWS, ew), 0)
               ).astype(jnp.bfloat16)
        roh = (jnp.broadcast_to(rv, (LANE, ew))
               == lax.broadcasted_iota(jnp.int32, (LANE, ew), 0)
               ).astype(jnp.bfloat16)
        p = lax.dot_general(qoh, roh, (((1,), (1,)), ((), ())),
                            preferred_element_type=jnp.float32)
        accq[...] = jnp.where(i == 0, p, accq[...] + p)

    @pl.when(i >= _ESTEPS)
    def _():
        j = i - _ESTEPS
        degblk = accq[pl.ds(j * (_R // LANE), _R // LANE), :]
        rep = (lax.broadcasted_iota(jnp.int32, (_R, _R // LANE), 0) // LANE
               == lax.broadcasted_iota(jnp.int32, (_R, _R // LANE), 1)
               ).astype(jnp.float32)
        rows = lax.dot_general(rep, degblk, (((1,), (0,)), ((), ())),
                               preferred_element_type=jnp.float32)
        lanemask = (lax.broadcasted_iota(jnp.int32, (_R, LANE), 0) % LANE
                    == lax.broadcasted_iota(jnp.int32, (_R, LANE), 1))
        deg = jnp.sum(jnp.where(lanemask, rows, 0.0), axis=1) + 1.0
        dv = lax.rsqrt(jnp.maximum(deg, 1e-12))
        dinv_ref[0, 0, :] = dv
        dinv = dv[:, None]
        xb = x_ref[...]
        ha = jnp.dot(xb, wa_ref[...], preferred_element_type=jnp.float32)
        hc = jnp.dot(xb, wc_ref[...], preferred_element_type=jnp.float32)
        g_ref[0] = ha[:, :LANE] * dinv
        g_ref[1] = ha[:, LANE:] * dinv
        g_ref[2] = hc[:, :LANE] * dinv
        g_ref[3] = hc[:, LANE:] * dinv


def _k1_call(x, wa, wc, dst_rows):
    nsteps, _, ew = dst_rows.shape
    assert nsteps == _ESTEPS
    nj = lambda i: jnp.maximum(i - _ESTEPS, 0)
    return pl.pallas_call(
        functools.partial(_k1_body, ew),
        grid=(_ESTEPS + _G,),
        in_specs=[
            pl.BlockSpec((1, 1, ew), lambda i: (jnp.minimum(i, _ESTEPS - 1), 0, 0)),
            pl.BlockSpec((_R, D), lambda i: (nj(i), 0)),
            pl.BlockSpec((D, D), lambda i: (0, 0)),
            pl.BlockSpec((D, D), lambda i: (0, 0)),
        ],
        out_specs=[
            pl.BlockSpec((4, _R, LANE), lambda i: (0, nj(i), 0)),
            pl.BlockSpec((1, 1, _R), lambda i: (nj(i), 0, 0)),
        ],
        out_shape=[
            jax.ShapeDtypeStruct((4, NT, LANE), jnp.float32),
            jax.ShapeDtypeStruct((_G, 1, _R), jnp.float32),
        ],
        scratch_shapes=[pltpu.VMEM((_QROWS, LANE), jnp.float32)],
    )(dst_rows, x, wa, wc)


def _sigmoid(x):
    return 1.0 / (1.0 + jnp.exp(-x))


def _softplus(x):
    return jnp.maximum(x, 0.0) + jnp.log1p(jnp.exp(-jnp.abs(x)))


def _k2_body(s_ref, dinv_ref, x_ref, ba_ref, bc_ref,
             w1a_ref, b1a_ref, w2a_ref, b2a_ref, w3a_ref, b3a_ref,
             w1c_ref, b1c_ref, w2c_ref, b2c_ref, w3c_ref, b3c_ref,
             conc_ref, nz_ref, val_ref, pooled):
    i = pl.program_id(0)
    dinv = dinv_ref[0, 0, :][:, None]
    xb = x_ref[...]
    valid2 = (i * _R + lax.broadcasted_iota(jnp.int32, (_R, 1), 0)) < N

    sa = jnp.concatenate([s_ref[0], s_ref[1]], axis=1)
    ha = jnp.maximum(sa * dinv + ba_ref[...], 0.0) + xb
    t = jnp.maximum(jnp.dot(ha, w1a_ref[...], preferred_element_type=jnp.float32)
                    + b1a_ref[...], 0.0)
    t = jnp.maximum(jnp.dot(t, w2a_ref[...], preferred_element_type=jnp.float32)
                    + b2a_ref[...], 0.0)
    lg = jnp.dot(t, w3a_ref[...], preferred_element_type=jnp.float32) + b3a_ref[...]
    conc_ref[0, 0, :] = _softplus(lg[:, 0]) + JITTER
    nz_ref[0, 0, :] = _sigmoid(lg[:, 1])

    sc = jnp.concatenate([s_ref[2], s_ref[3]], axis=1)
    hc = jnp.maximum(sc * dinv + bc_ref[...], 0.0) + xb
    ps = jnp.sum(jnp.where(valid2, hc, 0.0), axis=0)
    pooled[0, :] = jnp.where(i == 0, ps, pooled[0, :] + ps)

    @pl.when(i == _G - 1)
    def _():
        p = pooled[0, :][None, :]
        u = jnp.maximum(jnp.dot(p, w1c_ref[...], preferred_element_type=jnp.float32)
                        + b1c_ref[...], 0.0)
        u = jnp.maximum(jnp.dot(u, w2c_ref[...], preferred_element_type=jnp.float32)
                        + b2c_ref[...], 0.0)
        v = jnp.dot(u, w3c_ref[...], preferred_element_type=jnp.float32)
        val_ref[...] = v[:, :1] + b3c_ref[...]


def _k2_call(s4, dinv5, x, ba, bc, w1a, b1a, w2a, b2a, w3ap, b3ap,
             w1c, b1c, w2c, b2c, w3cp, b3c2):
    full = lambda shape: pl.BlockSpec(shape, lambda i: tuple(0 for _ in shape))
    return pl.pallas_call(
        _k2_body,
        grid=(_G,),
        in_specs=[
            pl.BlockSpec((4, _R, LANE), lambda i: (0, i, 0)),
            pl.BlockSpec((1, 1, _R), lambda i: (i, 0, 0)),
            pl.BlockSpec((_R, D), lambda i: (i, 0)),
            full((D,)), full((D,)),
            full((D, 128)), full((128,)), full((128, 64)), full((64,)),
            full((64, 128)), full((128,)),
            full((D, 128)), full((128,)), full((128, 64)), full((64,)),
            full((64, 128)), full((1, 1)),
        ],
        out_specs=[
            pl.BlockSpec((1, 1, _R), lambda i: (i, 0, 0)),
            pl.BlockSpec((1, 1, _R), lambda i: (i, 0, 0)),
            pl.BlockSpec((1, 1), lambda i: (0, 0)),
        ],
        out_shape=[
            jax.ShapeDtypeStruct((_G, 1, _R), jnp.float32),
            jax.ShapeDtypeStruct((_G, 1, _R), jnp.float32),
            jax.ShapeDtypeStruct((1, 1), jnp.float32),
        ],
        scratch_shapes=[pltpu.VMEM((1, D), jnp.float32)],
    )(s4, dinv5, x, ba, bc, w1a, b1a, w2a, b2a, w3ap, b3ap,
      w1c, b1c, w2c, b2c, w3cp, b3c2)


def kernel(x, edge_index, W_gcn_a, b_gcn_a, W1a, b1a, W2a, b2a, W3a, b3a,
           W_gcn_c, b_gcn_c, W1c, b1c, W2c, b2c, W3c, b3c):
    E = edge_index.shape[1]
    ep, nch = _edge_layout(E)
    pad = ep - E

    ei = edge_index.astype(jnp.int32)
    apad = jnp.arange(pad, dtype=jnp.int32)
    src_p = jnp.concatenate([ei[0], apad % N])
    dst_p = jnp.concatenate([ei[1], N + apad % TRASH])
    srcs = (src_p[None, :]
            + (jnp.arange(4, dtype=jnp.int32) * NT)[:, None]).reshape(-1, CHUNK)
    dst2d = dst_p.reshape(nch, CHUNK)

    g4, dinv5 = _k1_call(x, W_gcn_a, W_gcn_c, dst_p.reshape(_ESTEPS, 1, -1))
    g4 = g4.reshape(4 * NT, LANE)
    s4 = _scatter_call(g4, srcs, dst2d, nch).reshape(4, N, LANE)

    w3ap = jnp.pad(W3a, ((0, 0), (0, 126)))
    b3ap = jnp.pad(b3a, (0, 126))
    w3cp = jnp.pad(W3c, ((0, 0), (0, 127)))
    b3c2 = b3c.reshape(1, 1)
    conc2, nz2, val = _k2_call(
        s4, dinv5, x, b_gcn_a, b_gcn_c, W1a, b1a, W2a, b2a, w3ap, b3ap,
        W1c, b1c, W2c, b2c, w3cp, b3c2)
    return conc2.reshape(-1)[:N], nz2.reshape(-1)[:N], val.reshape(-1)


# speedup vs baseline: 1.0261x; 1.0220x over previous
import functools

import jax
import jax.numpy as jnp
from jax import lax
from jax.experimental import pallas as pl
from jax.experimental.pallas import tpu as pltpu
from jax.experimental.pallas import tpu_sc as plsc

N = 10000
D = 256
LANE = 128
CHUNK = 128
TILES = 16
NP = 10240
NPB = 10048
TRASH = NPB - N
NT = NP

JITTER = 1e-20


def _edge_layout(E):
    ep = ((E + 32 * CHUNK - 1) // (32 * CHUNK)) * (32 * CHUNK)
    return ep, ep // CHUNK


_QROWS = NP // LANE


def _scatter_body(nch_t, g4_hbm, srcs_hbm, dst_hbm, out_hbm,
                  src_v, dst_v, buf0, buf1, acc, sem0, sem1):
    cid = lax.axis_index("c")
    sid = lax.axis_index("s")
    rows_t = 624
    tail = N - TILES * rows_t

    hb = nch_t // 2
    SUBS = ((0, 160), (160, 160), (320, 160), (480, 144))

    def init_phase(ph):
        pltpu.sync_copy(g4_hbm.at[pl.ds(ph * NT + sid * rows_t, rows_t)],
                        acc.at[pl.ds(sid * rows_t, rows_t)])

        @pl.when(sid == TILES - 1)
        def _():
            pltpu.sync_copy(
                g4_hbm.at[pl.ds(ph * NT + TILES * rows_t, tail)],
                acc.at[pl.ds(TILES * rows_t, tail)])

    def readout_phase(ph):
        pltpu.sync_copy(acc.at[pl.ds(sid * rows_t, rows_t)],
                        out_hbm.at[pl.ds(ph * N + sid * rows_t, rows_t)])

        @pl.when(sid == TILES - 1)
        def _():
            pltpu.sync_copy(
                acc.at[pl.ds(TILES * rows_t, tail)],
                out_hbm.at[pl.ds(ph * N + TILES * rows_t, tail)])

    def boundary(ph_out, ph_in, rsem, isem):
        def ro(k, start):
            off, ln = SUBS[k]
            d = pltpu.make_async_copy(
                acc.at[pl.ds(sid * rows_t + off, ln)],
                out_hbm.at[pl.ds(ph_out * N + sid * rows_t + off, ln)], rsem)
            d.start() if start else d.wait()

        def ini(k, start):
            off, ln = SUBS[k]
            d = pltpu.make_async_copy(
                g4_hbm.at[pl.ds(ph_in * NT + sid * rows_t + off, ln)],
                acc.at[pl.ds(sid * rows_t + off, ln)], isem)
            d.start() if start else d.wait()

        ro(0, True)
        for k in range(len(SUBS)):
            ro(k, False)
            if k + 1 < len(SUBS):
                ro(k + 1, True)
            ini(k, True)

        @pl.when(sid == TILES - 1)
        def _():
            pltpu.sync_copy(
                acc.at[pl.ds(TILES * rows_t, tail)],
                out_hbm.at[pl.ds(ph_out * N + TILES * rows_t, tail)])
            pltpu.sync_copy(
                g4_hbm.at[pl.ds(ph_in * NT + TILES * rows_t, tail)],
                acc.at[pl.ds(TILES * rows_t, tail)])

        for k in range(len(SUBS)):
            ini(k, False)

    for h in range(2):
        ph = h * 2 + cid
        if h == 0:
            init_phase(ph)

        for half in range(2):
            base = sid * nch_t + half * hb
            pltpu.sync_copy(srcs_hbm.at[pl.ds(ph * (TILES * nch_t) + base, hb)],
                            src_v)
            pltpu.sync_copy(dst_hbm.at[pl.ds(base, hb)], dst_v)
            if half == 0:
                plsc.subcore_barrier()

            def gather(j, buf, sem):
                pltpu.async_copy(g4_hbm.at[src_v.at[j]], buf, sem)

            def gwait(j, buf, sem):
                pltpu.make_async_copy(g4_hbm.at[src_v.at[j]], buf, sem).wait()

            gather(0, buf0, sem0)
            gather(1, buf1, sem1)

            @pl.loop(0, hb // 2)
            def _(t):
                j = 2 * t
                gwait(j, buf0, sem0)
                pltpu.sync_copy(buf0, acc.at[dst_v.at[j]], add=True)

                @pl.when(j + 2 < hb)
                def _():
                    gather(j + 2, buf0, sem0)

                gwait(j + 1, buf1, sem1)
                pltpu.sync_copy(buf1, acc.at[dst_v.at[j + 1]], add=True)

                @pl.when(j + 3 < hb)
                def _():
                    gather(j + 3, buf1, sem1)

        plsc.subcore_barrier()
        if h == 0:
            boundary(ph, 1 * 2 + cid, sem0, sem1)
            plsc.subcore_barrier()
        else:
            readout_phase(ph)


def _scatter_call(g4, srcs, dst2d, nch):
    nch_t = nch // TILES
    mesh = plsc.VectorSubcoreMesh(core_axis_name="c", subcore_axis_name="s")
    f = functools.partial(
        pl.kernel,
        out_type=jax.ShapeDtypeStruct((4 * N, LANE), jnp.float32),
        mesh=mesh,
        scratch_types=[
            pltpu.VMEM((nch_t // 2, CHUNK), jnp.int32),
            pltpu.VMEM((nch_t // 2, CHUNK), jnp.int32),
            pltpu.VMEM((CHUNK, LANE), jnp.float32),
            pltpu.VMEM((CHUNK, LANE), jnp.float32),
            pltpu.VMEM_SHARED((NPB, LANE), jnp.float32),
            pltpu.SemaphoreType.DMA,
            pltpu.SemaphoreType.DMA,
        ],
    )(functools.partial(_scatter_body, nch_t))
    return f(g4, srcs, dst2d)


_R = 2048
_G = NT // _R
_ESTEPS = 16


def _k1_body(ew, dst_ref, x_ref, wa_ref, wc_ref, g_ref, dinv_ref, accq):
    i = pl.program_id(0)

    @pl.when(i < _ESTEPS)
    def _():
        d = dst_ref[...][0]
        qv = d // LANE
        rv = d % LANE
        qoh = (jnp.broadcast_to(qv, (_QROWS, ew))
               == lax.broadcasted_iota(jnp.int32, (_QROWS, ew), 0)
               ).astype(jnp.bfloat16)
        roh = (jnp.broadcast_to(rv, (LANE, ew))
               == lax.broadcasted_iota(jnp.int32, (LANE, ew), 0)
               ).astype(jnp.bfloat16)
        p = lax.dot_general(qoh, roh, (((1,), (1,)), ((), ())),
                            preferred_element_type=jnp.float32)
        accq[...] = jnp.where(i == 0, p, accq[...] + p)

    @pl.when(i >= _ESTEPS)
    def _():
        j = i - _ESTEPS
        degblk = accq[pl.ds(j * (_R // LANE), _R // LANE), :]
        rep = (lax.broadcasted_iota(jnp.int32, (_R, _R // LANE), 0) // LANE
               == lax.broadcasted_iota(jnp.int32, (_R, _R // LANE), 1)
               ).astype(jnp.float32)
        rows = lax.dot_general(rep, degblk, (((1,), (0,)), ((), ())),
                               preferred_element_type=jnp.float32)
        lanemask = (lax.broadcasted_iota(jnp.int32, (_R, LANE), 0) % LANE
                    == lax.broadcasted_iota(jnp.int32, (_R, LANE), 1))
        deg = jnp.sum(jnp.where(lanemask, rows, 0.0), axis=1) + 1.0
        dv = lax.rsqrt(jnp.maximum(deg, 1e-12))
        dinv_ref[0, 0, :] = dv
        dinv = dv[:, None]
        xb = x_ref[...]
        ha = jnp.dot(xb, wa_ref[...], preferred_element_type=jnp.float32)
        hc = jnp.dot(xb, wc_ref[...], preferred_element_type=jnp.float32)
        g_ref[0] = ha[:, :LANE] * dinv
        g_ref[1] = ha[:, LANE:] * dinv
        g_ref[2] = hc[:, :LANE] * dinv
        g_ref[3] = hc[:, LANE:] * dinv


def _k1_call(x, wa, wc, dst_rows):
    nsteps, _, ew = dst_rows.shape
    assert nsteps == _ESTEPS
    nj = lambda i: jnp.maximum(i - _ESTEPS, 0)
    return pl.pallas_call(
        functools.partial(_k1_body, ew),
        grid=(_ESTEPS + _G,),
        in_specs=[
            pl.BlockSpec((1, 1, ew), lambda i: (jnp.minimum(i, _ESTEPS - 1), 0, 0)),
            pl.BlockSpec((_R, D), lambda i: (nj(i), 0)),
            pl.BlockSpec((D, D), lambda i: (0, 0)),
            pl.BlockSpec((D, D), lambda i: (0, 0)),
        ],
        out_specs=[
            pl.BlockSpec((4, _R, LANE), lambda i: (0, nj(i), 0)),
            pl.BlockSpec((1, 1, _R), lambda i: (nj(i), 0, 0)),
        ],
        out_shape=[
            jax.ShapeDtypeStruct((4, NT, LANE), jnp.float32),
            jax.ShapeDtypeStruct((_G, 1, _R), jnp.float32),
        ],
        scratch_shapes=[pltpu.VMEM((_QROWS, LANE), jnp.float32)],
    )(dst_rows, x, wa, wc)


def _sigmoid(x):
    return 1.0 / (1.0 + jnp.exp(-x))


def _softplus(x):
    return jnp.maximum(x, 0.0) + jnp.log1p(jnp.exp(-jnp.abs(x)))


def _k2_body(s_ref, dinv_ref, x_ref, ba_ref, bc_ref,
             w1a_ref, b1a_ref, w2a_ref, b2a_ref, w3a_ref, b3a_ref,
             w1c_ref, b1c_ref, w2c_ref, b2c_ref, w3c_ref, b3c_ref,
             conc_ref, nz_ref, val_ref, pooled):
    i = pl.program_id(0)
    dinv = dinv_ref[0, 0, :][:, None]
    xb = x_ref[...]
    valid2 = (i * _R + lax.broadcasted_iota(jnp.int32, (_R, 1), 0)) < N

    sa = jnp.concatenate([s_ref[0], s_ref[1]], axis=1)
    ha = jnp.maximum(sa * dinv + ba_ref[...], 0.0) + xb
    t = jnp.maximum(jnp.dot(ha, w1a_ref[...], preferred_element_type=jnp.float32)
                    + b1a_ref[...], 0.0)
    t = jnp.maximum(jnp.dot(t, w2a_ref[...], preferred_element_type=jnp.float32)
                    + b2a_ref[...], 0.0)
    lg = jnp.dot(t, w3a_ref[...], preferred_element_type=jnp.float32) + b3a_ref[...]
    conc_ref[0, 0, :] = _softplus(lg[:, 0]) + JITTER
    nz_ref[0, 0, :] = _sigmoid(lg[:, 1])

    sc = jnp.concatenate([s_ref[2], s_ref[3]], axis=1)
    hc = jnp.maximum(sc * dinv + bc_ref[...], 0.0) + xb
    ps = jnp.sum(jnp.where(valid2, hc, 0.0), axis=0)
    pooled[0, :] = jnp.where(i == 0, ps, pooled[0, :] + ps)

    @pl.when(i == _G - 1)
    def _():
        p = pooled[0, :][None, :]
        u = jnp.maximum(jnp.dot(p, w1c_ref[...], preferred_element_type=jnp.float32)
                        + b1c_ref[...], 0.0)
        u = jnp.maximum(jnp.dot(u, w2c_ref[...], preferred_element_type=jnp.float32)
                        + b2c_ref[...], 0.0)
        v = jnp.dot(u, w3c_ref[...], preferred_element_type=jnp.float32)
        val_ref[...] = v[:, :1] + b3c_ref[...]


def _k2_call(s4, dinv5, x, ba, bc, w1a, b1a, w2a, b2a, w3ap, b3ap,
             w1c, b1c, w2c, b2c, w3cp, b3c2):
    full = lambda shape: pl.BlockSpec(shape, lambda i: tuple(0 for _ in shape))
    return pl.pallas_call(
        _k2_body,
        grid=(_G,),
        in_specs=[
            pl.BlockSpec((4, _R, LANE), lambda i: (0, i, 0)),
            pl.BlockSpec((1, 1, _R), lambda i: (i, 0, 0)),
            pl.BlockSpec((_R, D), lambda i: (i, 0)),
            full((D,)), full((D,)),
            full((D, 128)), full((128,)), full((128, 64)), full((64,)),
            full((64, 128)), full((128,)),
            full((D, 128)), full((128,)), full((128, 64)), full((64,)),
            full((64, 128)), full((1, 1)),
        ],
        out_specs=[
            pl.BlockSpec((1, 1, _R), lambda i: (i, 0, 0)),
            pl.BlockSpec((1, 1, _R), lambda i: (i, 0, 0)),
            pl.BlockSpec((1, 1), lambda i: (0, 0)),
        ],
        out_shape=[
            jax.ShapeDtypeStruct((_G, 1, _R), jnp.float32),
            jax.ShapeDtypeStruct((_G, 1, _R), jnp.float32),
            jax.ShapeDtypeStruct((1, 1), jnp.float32),
        ],
        scratch_shapes=[pltpu.VMEM((1, D), jnp.float32)],
    )(s4, dinv5, x, ba, bc, w1a, b1a, w2a, b2a, w3ap, b3ap,
      w1c, b1c, w2c, b2c, w3cp, b3c2)


def kernel(x, edge_index, W_gcn_a, b_gcn_a, W1a, b1a, W2a, b2a, W3a, b3a,
           W_gcn_c, b_gcn_c, W1c, b1c, W2c, b2c, W3c, b3c):
    E = edge_index.shape[1]
    ep, nch = _edge_layout(E)
    pad = ep - E

    ei = edge_index.astype(jnp.int32)
    apad = jnp.arange(pad, dtype=jnp.int32)
    src_p = jnp.concatenate([ei[0], apad % N])
    dst_p = jnp.concatenate([ei[1], N + apad % TRASH])
    srcs = (src_p[None, :]
            + (jnp.arange(4, dtype=jnp.int32) * NT)[:, None]).reshape(-1, CHUNK)
    dst2d = dst_p.reshape(nch, CHUNK)

    g4, dinv5 = _k1_call(x, W_gcn_a, W_gcn_c, dst_p.reshape(_ESTEPS, 1, -1))
    g4 = g4.reshape(4 * NT, LANE)
    s4 = _scatter_call(g4, srcs, dst2d, nch).reshape(4, N, LANE)

    w3ap = jnp.pad(W3a, ((0, 0), (0, 126)))
    b3ap = jnp.pad(b3a, (0, 126))
    w3cp = jnp.pad(W3c, ((0, 0), (0, 127)))
    b3c2 = b3c.reshape(1, 1)
    conc2, nz2, val = _k2_call(
        s4, dinv5, x, b_gcn_a, b_gcn_c, W1a, b1a, W2a, b2a, w3ap, b3ap,
        W1c, b1c, W2c, b2c, w3cp, b3c2)
    return conc2.reshape(-1)[:N], nz2.reshape(-1)[:N], val.reshape(-1)
